# baseline (speedup 1.0000x reference)
import functools

import jax
import jax.numpy as jnp
from jax import lax
from jax.experimental import pallas as pl
from jax.experimental.pallas import tpu as pltpu
from jax.experimental.pallas import tpu_sc as plsc

U_A = 110.0

_SC_CORES = 2
_SC_SUBCORES = 16
_LANES = 16


def _sc_gather_body(rpw, v, ctm_hbm, tgt_hbm, vals_hbm, tgt_v, win_v,
                    val_v, sem):
    wid = lax.axis_index("s") * _SC_CORES + lax.axis_index("c")
    base = pl.multiple_of(wid * rpw, rpw)
    pltpu.sync_copy(tgt_hbm.at[pl.ds(base, rpw)], tgt_v)
    lane_iota = lax.iota(jnp.int32, _LANES)
    copies = []
    for j in range(rpw):
        chunk, lane = divmod(j, _LANES)
        t16 = tgt_v[pl.ds(chunk * _LANES, _LANES)]
        tj = jnp.max(jnp.where(lane_iota == lane, t16, jnp.int32(-1)))
        kc = pl.multiple_of((tj >> 7) << 7, 128)
        r0 = base + (j & ~7)
        copies.append(
            pltpu.async_copy(ctm_hbm.at[pl.ds(r0, 8), pl.ds(kc, 128)],
                             win_v.at[j], sem))
    for cp in copies:
        cp.wait()
    for chunk in range(rpw // _LANES):
        rows16 = lax.iota(jnp.int32, _LANES) + chunk * _LANES
        subs16 = jnp.bitwise_and(rows16, jnp.int32(7))
        t16 = tgt_v[pl.ds(chunk * _LANES, _LANES)]
        lanes16 = jnp.bitwise_and(t16, jnp.int32(127))
        val_v[pl.ds(chunk * _LANES, _LANES)] = plsc.load_gather(
            win_v, [rows16, subs16, lanes16])
    pltpu.sync_copy(val_v, vals_hbm.at[pl.ds(base, rpw)])


def _sc_gather(ctm, target):
    b, v = ctm.shape
    nw = _SC_CORES * _SC_SUBCORES
    rpw = b // nw
    mesh = plsc.VectorSubcoreMesh(core_axis_name="c", subcore_axis_name="s")
    fn = functools.partial(
        pl.kernel,
        out_type=jax.ShapeDtypeStruct((b,), jnp.float32),
        mesh=mesh,
        scratch_types=[
            pltpu.VMEM((rpw,), jnp.int32),
            pltpu.VMEM((rpw, 8, 128), jnp.float32),
            pltpu.VMEM((rpw,), jnp.float32),
            pltpu.SemaphoreType.DMA,
        ],
        compiler_params=pltpu.CompilerParams(use_tc_tiling_on_sc=True,
                                             needs_layout_passes=False),
    )(functools.partial(_sc_gather_body, rpw, v))
    return fn(ctm, target)


def _mag_body(num_blocks, bv, v, tgt_ref, vm_ref, xn_ref, ct_ref,
              out_ref, loss_ref, lossg_ref, m_ref, s_ref):
    j = pl.program_id(0)
    b = ct_ref.shape[0]

    @pl.when(j == 0)
    def _init():
        m_ref[...] = jnp.full_like(m_ref, -jnp.inf)
        s_ref[...] = jnp.zeros_like(s_ref)

    c = ct_ref[...]
    cols = j * bv + lax.broadcasted_iota(jnp.int32, (b, bv), 1)
    t = tgt_ref[...]
    blk = jnp.where(cols == t, vm_ref[...], c)
    out_ref[...] = blk

    valid = cols < v
    mblk = jnp.where(valid, blk, -jnp.inf)
    bm = jnp.max(mblk, axis=1, keepdims=True)
    m_old = m_ref[...]
    m_new = jnp.maximum(m_old, bm)
    e = jnp.exp(mblk - m_new)
    s_new = s_ref[...] * jnp.exp(m_old - m_new) + jnp.sum(e, axis=1,
                                                          keepdims=True)
    m_ref[...] = m_new
    s_ref[...] = s_new

    @pl.when(j == num_blocks - 1)
    def _finish():
        log_z = m_new + jnp.log(s_new)
        picked = vm_ref[...]
        loss_ref[...] = (jnp.sum(log_z - picked) / b).reshape(1, 1)
        xn = xn_ref[...]
        lossg_ref[...] = (jnp.sum(xn * (1.0 / (U_A * U_A)) + 1.0 / xn)
                          / b).reshape(1, 1)


def _mag_loss_tc(cos_theta, target, vals_m, x_norm, bv=2048):
    b, v = cos_theta.shape
    num_blocks = pl.cdiv(v, bv)
    grid = (num_blocks,)
    kernel_fn = functools.partial(_mag_body, num_blocks, bv, v)
    out, loss, loss_g = pl.pallas_call(
        kernel_fn,
        grid=grid,
        in_specs=[
            pl.BlockSpec((b, 1), lambda j: (0, 0)),
            pl.BlockSpec((b, 1), lambda j: (0, 0)),
            pl.BlockSpec((b, 1), lambda j: (0, 0)),
            pl.BlockSpec((b, bv), lambda j: (0, j)),
        ],
        out_specs=[
            pl.BlockSpec((b, bv), lambda j: (0, j)),
            pl.BlockSpec((1, 1), lambda j: (0, 0)),
            pl.BlockSpec((1, 1), lambda j: (0, 0)),
        ],
        out_shape=[
            jax.ShapeDtypeStruct((b, v), jnp.float32),
            jax.ShapeDtypeStruct((1, 1), jnp.float32),
            jax.ShapeDtypeStruct((1, 1), jnp.float32),
        ],
        scratch_shapes=[
            pltpu.VMEM((b, 1), jnp.float32),
            pltpu.VMEM((b, 1), jnp.float32),
        ],
    )(target[:, None], vals_m[:, None], x_norm[:, None], cos_theta)
    return out, loss[0, 0], loss_g[0, 0]


def kernel(cos_theta, cos_theta_m, target, x_norm):
    b, v = cos_theta.shape
    vals_m = _sc_gather(cos_theta_m, target)
    out, loss, loss_g = _mag_loss_tc(cos_theta, target, vals_m, x_norm,
                                     bv=2560)
    return (loss, loss_g, out)


# speedup vs baseline: 4.8676x; 4.8676x over previous
import functools

import jax
import jax.numpy as jnp
from jax import lax
from jax.experimental import pallas as pl
from jax.experimental.pallas import tpu as pltpu
from jax.experimental.pallas import tpu_sc as plsc

U_A = 110.0

_SC_CORES = 2
_SC_SUBCORES = 16
_LANES = 16


def _mag_body(num_blocks, bv, v, tgt_ref, vm_ref, xn_ref, ct_ref,
              out_ref, loss_ref, lossg_ref, m_ref, s_ref):
    j = pl.program_id(0)
    b = ct_ref.shape[1]

    @pl.when(j == 0)
    def _init():
        m_ref[...] = jnp.full_like(m_ref, -jnp.inf)
        s_ref[...] = jnp.zeros_like(s_ref)

    c = ct_ref[...]
    rows = j * bv + lax.broadcasted_iota(jnp.int32, (bv, b), 0)
    t = tgt_ref[...]
    blk = jnp.where(rows == t, vm_ref[...], c)
    out_ref[...] = blk

    valid = rows < v
    mblk = jnp.where(valid, blk, -jnp.inf)
    bm = jnp.max(mblk, axis=0, keepdims=True)
    m_old = m_ref[...]
    m_new = jnp.maximum(m_old, bm)
    e = jnp.exp(mblk - m_new)
    s_new = s_ref[...] * jnp.exp(m_old - m_new) + jnp.sum(e, axis=0,
                                                          keepdims=True)
    m_ref[...] = m_new
    s_ref[...] = s_new

    @pl.when(j == num_blocks - 1)
    def _finish():
        log_z = m_new + jnp.log(s_new)
        picked = vm_ref[...]
        loss_ref[...] = (jnp.sum(log_z - picked) / b).reshape(1, 1)
        xn = xn_ref[...]
        lossg_ref[...] = (jnp.sum(xn * (1.0 / (U_A * U_A)) + 1.0 / xn)
                          / b).reshape(1, 1)


def _mag_loss_tc(ct_t, target, vals_m, x_norm, bv=2048):
    v, b = ct_t.shape
    num_blocks = pl.cdiv(v, bv)
    kernel_fn = functools.partial(_mag_body, num_blocks, bv, v)
    out_t, loss, loss_g = pl.pallas_call(
        kernel_fn,
        grid=(num_blocks,),
        in_specs=[
            pl.BlockSpec((1, b), lambda j: (0, 0)),
            pl.BlockSpec((1, b), lambda j: (0, 0)),
            pl.BlockSpec((1, b), lambda j: (0, 0)),
            pl.BlockSpec((bv, b), lambda j: (j, 0)),
        ],
        out_specs=[
            pl.BlockSpec((bv, b), lambda j: (j, 0)),
            pl.BlockSpec((1, 1), lambda j: (0, 0)),
            pl.BlockSpec((1, 1), lambda j: (0, 0)),
        ],
        out_shape=[
            jax.ShapeDtypeStruct((v, b), jnp.float32),
            jax.ShapeDtypeStruct((1, 1), jnp.float32),
            jax.ShapeDtypeStruct((1, 1), jnp.float32),
        ],
        scratch_shapes=[
            pltpu.VMEM((1, b), jnp.float32),
            pltpu.VMEM((1, b), jnp.float32),
        ],
    )(target[None, :], vals_m[None, :], x_norm[None, :], ct_t)
    return out_t, loss[0, 0], loss_g[0, 0]


def kernel(cos_theta, cos_theta_m, target, x_norm):
    ct_t = jnp.swapaxes(cos_theta, 0, 1)
    vals_m = jnp.take_along_axis(cos_theta_m, target[:, None], axis=1)[:, 0]
    out_t, loss, loss_g = _mag_loss_tc(ct_t, target, vals_m, x_norm)
    return (loss, loss_g, jnp.swapaxes(out_t, 0, 1))
